# baseline (speedup 1.0000x reference)
import functools

import jax
import jax.numpy as jnp
from jax import lax
from jax.experimental import pallas as pl
from jax.experimental.pallas import tpu as pltpu
from jax.experimental.pallas import tpu_sc as plsc

N = 10000
E = 320000
D = 128

NC = 2
NS = 16
NW = NC * NS
L = 16

N_PAD = 10240
C = 128
E_PAD = 327680
PAGE = 8
EPP = PAGE * C
NPAGE_TOT = E_PAD // (NS * EPP)
P0 = 19
P1 = NPAGE_TOT - P0
ZROWS = N_PAD // NS
OROWS = N // NS

def _sc_aggregate_body(x_hbm, src_hbm, dst_hbm, parts_hbm, cnt_hbm,
                       sidx, didx, rows, ones_v, zv, acc, cacc,
                       gsem0, gsem1, ssem0, ssem1, psem, *, with_cnt):
    cid = lax.axis_index("c")
    sid = lax.axis_index("s")
    wid = sid * NC + cid
    z16 = jnp.zeros((L,), jnp.float32)
    o16 = jnp.ones((L,), jnp.float32)
    gsems = (gsem0, gsem1)
    ssems = (ssem0, ssem1)
    npage = NPAGE_TOT
    pbase = sid * NPAGE_TOT

    def load_page(p, pb):
        row0 = pl.multiple_of((pbase + p) * PAGE, PAGE)
        off = pl.multiple_of(row0 * C, C)
        pltpu.async_copy(src_hbm.at[pl.ds(off, EPP)], sidx.at[pb], psem)
        pltpu.async_copy(dst_hbm.at[pl.ds(row0, PAGE)], didx.at[pb], psem)

    def wait_page(pb):
        pltpu.make_async_copy(src_hbm.at[pl.ds(0, EPP)], sidx.at[pb],
                              psem).wait()
        pltpu.make_async_copy(dst_hbm.at[pl.ds(0, PAGE)], didx.at[pb],
                              psem).wait()

    def fire_gather(pb, c, rb):
        off = pl.multiple_of(c * C, C)
        pltpu.async_copy(x_hbm.at[sidx.at[pb, pl.ds(off, C)]], rows.at[rb],
                         gsems[rb])

    def wait_gather(rb):
        pltpu.make_async_copy(x_hbm.at[pl.ds(0, C)], rows.at[rb],
                              gsems[rb]).wait()

    DIAG_NO_SCATTER = False

    def fire_scatter(pb, c, rb):
        if DIAG_NO_SCATTER:
            return
        didx_c = didx.at[pb, c]
        pltpu.async_copy(rows.at[rb], acc.at[didx_c], ssems[rb], add=True)
        if with_cnt:
            pltpu.async_copy(ones_v, cacc.at[didx_c], ssems[rb], add=True)

    def wait_scatter(rb):
        if DIAG_NO_SCATTER:
            return
        pltpu.make_async_copy(rows.at[rb], acc.at[pl.ds(0, C)],
                              ssems[rb]).wait()
        if with_cnt:
            pltpu.make_async_copy(ones_v, cacc.at[pl.ds(0, C)],
                                  ssems[rb]).wait()

    load_page(0, 0)
    if with_cnt:
        def fill_small(i, _):
            ones_v[pl.ds(i * L, L)] = o16
            return _
        lax.fori_loop(0, C // L, fill_small, 0)

        def fill_zv(i, _):
            zv[pl.ds(i * L, L)] = z16
            return _
        lax.fori_loop(0, ZROWS // L, fill_zv, 0)
        pltpu.sync_copy(zv, cacc.at[pl.ds(sid * ZROWS, ZROWS)])

    def fill_rows(i, _):
        rows[0, i // (D // L), pl.ds((i % (D // L)) * L, L)] = z16
        return _
    lax.fori_loop(0, C * D // L, fill_rows, 0)
    for b in range(ZROWS // C):
        pltpu.sync_copy(rows.at[0], acc.at[pl.ds(sid * ZROWS + b * C, C)])

    wait_page(0)
    fire_gather(0, 0, 0)
    plsc.subcore_barrier()

    def page_body(p, _):
        pb = lax.rem(p, 2)
        last_page = p + 1 >= npage

        for c in range(PAGE):
            rb = c & 1
            wait_gather(rb)
            fire_scatter(pb, c, rb)
            if c == 0:
                @pl.when(p > 0)
                def _wprev():
                    wait_scatter(1 - rb)

                @pl.when(jnp.logical_not(last_page))
                def _prefetch():
                    load_page(p + 1, 1 - pb)
            else:
                wait_scatter(1 - rb)
            if c + 1 < PAGE:
                fire_gather(pb, c + 1, 1 - rb)
            else:
                @pl.when(jnp.logical_not(last_page))
                def _next():
                    wait_page(1 - pb)
                    fire_gather(1 - pb, 0, 1 - rb)
        return _
    lax.fori_loop(0, npage, page_body, 0)
    wait_scatter(1)
    plsc.subcore_barrier()

    pltpu.sync_copy(acc.at[pl.ds(sid * ZROWS, ZROWS)],
                    parts_hbm.at[cid, pl.ds(sid * ZROWS, ZROWS)])
    if with_cnt:
        pltpu.sync_copy(cacc.at[pl.ds(sid * ZROWS, ZROWS)],
                        cnt_hbm.at[pl.ds(pl.multiple_of(cid * N_PAD + sid * ZROWS, 8),
                                         ZROWS)])


@functools.cache
def _sc_aggregate(with_cnt):
    mesh = plsc.VectorSubcoreMesh(core_axis_name="c", subcore_axis_name="s",
                                  num_cores=1, num_subcores=NS)
    return pl.kernel(
        functools.partial(_sc_aggregate_body, with_cnt=with_cnt),
        out_type=(
            jax.ShapeDtypeStruct((1, N_PAD, D), jnp.float32),
            jax.ShapeDtypeStruct((N_PAD,), jnp.float32),
        ),
        mesh=mesh,
        scratch_types=[
            pltpu.VMEM((2, PAGE * C), jnp.int32),
            pltpu.VMEM((2, PAGE, C), jnp.int32),
            pltpu.VMEM((2, C, D), jnp.float32),
            pltpu.VMEM((C,), jnp.float32),
            pltpu.VMEM((ZROWS,), jnp.float32),
            pltpu.VMEM_SHARED((N_PAD, D), jnp.float32),
            pltpu.VMEM_SHARED((N_PAD,), jnp.float32),
            pltpu.SemaphoreType.DMA,
            pltpu.SemaphoreType.DMA,
            pltpu.SemaphoreType.DMA,
            pltpu.SemaphoreType.DMA,
            pltpu.SemaphoreType.DMA,
        ],
    )


BLK = 400


def _dense_body(parts_ref, inv_ref, x_ref, wlt_ref, bl_ref, wrt_ref, o_ref,
                *, act):
    summed = parts_ref[0]
    mean = summed * inv_ref[...]
    out = (jnp.dot(mean, wlt_ref[...], preferred_element_type=jnp.float32)
           + jnp.dot(x_ref[...], wrt_ref[...], preferred_element_type=jnp.float32)
           + bl_ref[...])
    o_ref[...] = jnp.tanh(out) if act else out


def _dense(parts, inv, x, Wl, bl, Wr, act):
    return pl.pallas_call(
        functools.partial(_dense_body, act=act),
        grid=(N // BLK,),
        in_specs=[
            pl.BlockSpec((1, BLK, D), lambda i: (0, i, 0)),
            pl.BlockSpec((BLK, 1), lambda i: (i, 0)),
            pl.BlockSpec((BLK, D), lambda i: (i, 0)),
            pl.BlockSpec((D, D), lambda i: (0, 0)),
            pl.BlockSpec((1, D), lambda i: (0, 0)),
            pl.BlockSpec((D, D), lambda i: (0, 0)),
        ],
        out_specs=pl.BlockSpec((BLK, D), lambda i: (i, 0)),
        out_shape=jax.ShapeDtypeStruct((N, D), jnp.float32),
    )(parts, inv, x, Wl.T, bl.reshape(1, D), Wr.T)


def kernel(x, edge_index, W1l, b1l, W1r, W2l, b2l, W2r):
    ei = edge_index.astype(jnp.int32)
    npad = E_PAD - E
    src_p = jnp.concatenate([ei[0], jnp.zeros((npad,), jnp.int32)])
    dst_p = jnp.concatenate(
        [ei[1], N + (jnp.arange(npad, dtype=jnp.int32) % (N_PAD - N))])
    dst_p = dst_p.reshape(E_PAD // C, C)

    parts1, cnt1 = _sc_aggregate(True)(x, src_p, dst_p)
    inv = (1.0 / jnp.maximum(cnt1[:N], 1.0)).reshape(N, 1)
    h = _dense(parts1, inv, x, W1l, b1l, W1r, True)
    parts2, _ = _sc_aggregate(False)(h, src_p, dst_p)
    return _dense(parts2, inv, h, W2l, b2l, W2r, False)


# speedup vs baseline: 1.4975x; 1.4975x over previous
import functools

import jax
import jax.numpy as jnp
from jax import lax
from jax.experimental import pallas as pl
from jax.experimental.pallas import tpu as pltpu
from jax.experimental.pallas import tpu_sc as plsc

N = 10000
E = 320000
D = 128

NC = 2
NS = 16
NW = NC * NS
L = 16

N_PAD = 10240
C = 128
E_PAD = 327680
PAGE = 8
EPP = PAGE * C
NPAGE_TOT = E_PAD // (NS * EPP)
P0 = 19
P1 = NPAGE_TOT - P0
ZROWS = N_PAD // NS
OROWS = N // NS

def _sc_aggregate_body(x_hbm, src_hbm, dst_hbm, parts_hbm, cnt_hbm,
                       sidx, didx, rows, ones_v, zv, acc, cacc,
                       gsem0, gsem1, ssem0, ssem1, psem, *, with_cnt):
    cid = lax.axis_index("c")
    sid = lax.axis_index("s")
    wid = sid * NC + cid
    z16 = jnp.zeros((L,), jnp.float32)
    o16 = jnp.ones((L,), jnp.float32)
    gsems = (gsem0, gsem1)
    ssems = (ssem0, ssem1)
    npage = jnp.where(cid == 0, P0, P1)
    pbase = jnp.where(cid == 0, sid * P0, NS * P0 + sid * P1)

    def load_page(p, pb):
        row0 = pl.multiple_of((pbase + p) * PAGE, PAGE)
        off = pl.multiple_of(row0 * C, C)
        pltpu.async_copy(src_hbm.at[pl.ds(off, EPP)], sidx.at[pb], psem)
        pltpu.async_copy(dst_hbm.at[pl.ds(row0, PAGE)], didx.at[pb], psem)

    def wait_page(pb):
        pltpu.make_async_copy(src_hbm.at[pl.ds(0, EPP)], sidx.at[pb],
                              psem).wait()
        pltpu.make_async_copy(dst_hbm.at[pl.ds(0, PAGE)], didx.at[pb],
                              psem).wait()

    def fire_gather(pb, c, rb):
        off = pl.multiple_of(c * C, C)
        pltpu.async_copy(x_hbm.at[sidx.at[pb, pl.ds(off, C)]], rows.at[rb],
                         gsems[rb])

    def wait_gather(rb):
        pltpu.make_async_copy(x_hbm.at[pl.ds(0, C)], rows.at[rb],
                              gsems[rb]).wait()

    DIAG_NO_SCATTER = False

    def fire_scatter(pb, c, rb):
        if DIAG_NO_SCATTER:
            return
        didx_c = didx.at[pb, c]
        pltpu.async_copy(rows.at[rb], acc.at[didx_c], ssems[rb], add=True)
        if with_cnt:
            pltpu.async_copy(ones_v, cacc.at[didx_c], ssems[rb], add=True)

    def wait_scatter(rb):
        if DIAG_NO_SCATTER:
            return
        pltpu.make_async_copy(rows.at[rb], acc.at[pl.ds(0, C)],
                              ssems[rb]).wait()
        if with_cnt:
            pltpu.make_async_copy(ones_v, cacc.at[pl.ds(0, C)],
                                  ssems[rb]).wait()

    load_page(0, 0)
    if with_cnt:
        def fill_small(i, _):
            ones_v[pl.ds(i * L, L)] = o16
            return _
        lax.fori_loop(0, C // L, fill_small, 0)

        def fill_zv(i, _):
            zv[pl.ds(i * L, L)] = z16
            return _
        lax.fori_loop(0, ZROWS // L, fill_zv, 0)
        pltpu.sync_copy(zv, cacc.at[pl.ds(sid * ZROWS, ZROWS)])

    def fill_rows(i, _):
        rows[0, i // (D // L), pl.ds((i % (D // L)) * L, L)] = z16
        return _
    lax.fori_loop(0, C * D // L, fill_rows, 0)
    for b in range(ZROWS // C):
        pltpu.sync_copy(rows.at[0], acc.at[pl.ds(sid * ZROWS + b * C, C)])

    wait_page(0)
    fire_gather(0, 0, 0)
    fire_gather(0, 1, 1)
    plsc.subcore_barrier()

    def page_body(p, _):
        pb = lax.rem(p, 2)
        last_page = p + 1 >= npage

        for c in range(PAGE):
            rb = c & 1
            wait_gather(rb)
            fire_scatter(pb, c, rb)
            wait_scatter(rb)
            if c == 0:
                @pl.when(jnp.logical_not(last_page))
                def _prefetch():
                    load_page(p + 1, 1 - pb)
            if c + 2 < PAGE:
                fire_gather(pb, c + 2, rb)
            else:
                if c + 2 == PAGE:
                    @pl.when(jnp.logical_not(last_page))
                    def _next0():
                        wait_page(1 - pb)
                        fire_gather(1 - pb, 0, rb)
                else:
                    @pl.when(jnp.logical_not(last_page))
                    def _next1():
                        fire_gather(1 - pb, 1, rb)
        return _
    lax.fori_loop(0, npage, page_body, 0)
    plsc.subcore_barrier()

    pltpu.sync_copy(acc.at[pl.ds(sid * ZROWS, ZROWS)],
                    parts_hbm.at[cid, pl.ds(sid * ZROWS, ZROWS)])
    if with_cnt:
        pltpu.sync_copy(cacc.at[pl.ds(sid * ZROWS, ZROWS)],
                        cnt_hbm.at[pl.ds(pl.multiple_of(cid * N_PAD + sid * ZROWS, 8),
                                         ZROWS)])


@functools.cache
def _sc_aggregate(with_cnt):
    mesh = plsc.VectorSubcoreMesh(core_axis_name="c", subcore_axis_name="s",
                                  num_cores=NC, num_subcores=NS)
    return pl.kernel(
        functools.partial(_sc_aggregate_body, with_cnt=with_cnt),
        out_type=(
            jax.ShapeDtypeStruct((NC, N_PAD, D), jnp.float32),
            jax.ShapeDtypeStruct((NC * N_PAD,), jnp.float32),
        ),
        mesh=mesh,
        scratch_types=[
            pltpu.VMEM((2, PAGE * C), jnp.int32),
            pltpu.VMEM((2, PAGE, C), jnp.int32),
            pltpu.VMEM((2, C, D), jnp.float32),
            pltpu.VMEM((C,), jnp.float32),
            pltpu.VMEM((ZROWS,), jnp.float32),
            pltpu.VMEM_SHARED((N_PAD, D), jnp.float32),
            pltpu.VMEM_SHARED((N_PAD,), jnp.float32),
            pltpu.SemaphoreType.DMA,
            pltpu.SemaphoreType.DMA,
            pltpu.SemaphoreType.DMA,
            pltpu.SemaphoreType.DMA,
            pltpu.SemaphoreType.DMA,
        ],
    )


BLK = 400


def _dense_body(parts_ref, inv_ref, x_ref, wlt_ref, bl_ref, wrt_ref, o_ref,
                *, act):
    summed = parts_ref[0] + parts_ref[1]
    mean = summed * inv_ref[...]
    out = (jnp.dot(mean, wlt_ref[...], preferred_element_type=jnp.float32)
           + jnp.dot(x_ref[...], wrt_ref[...], preferred_element_type=jnp.float32)
           + bl_ref[...])
    o_ref[...] = jnp.tanh(out) if act else out


def _dense(parts, inv, x, Wl, bl, Wr, act):
    return pl.pallas_call(
        functools.partial(_dense_body, act=act),
        grid=(N // BLK,),
        in_specs=[
            pl.BlockSpec((NC, BLK, D), lambda i: (0, i, 0)),
            pl.BlockSpec((BLK, 1), lambda i: (i, 0)),
            pl.BlockSpec((BLK, D), lambda i: (i, 0)),
            pl.BlockSpec((D, D), lambda i: (0, 0)),
            pl.BlockSpec((1, D), lambda i: (0, 0)),
            pl.BlockSpec((D, D), lambda i: (0, 0)),
        ],
        out_specs=pl.BlockSpec((BLK, D), lambda i: (i, 0)),
        out_shape=jax.ShapeDtypeStruct((N, D), jnp.float32),
    )(parts, inv, x, Wl.T, bl.reshape(1, D), Wr.T)


def kernel(x, edge_index, W1l, b1l, W1r, W2l, b2l, W2r):
    ei = edge_index.astype(jnp.int32)
    npad = E_PAD - E
    src_p = jnp.concatenate([ei[0], jnp.zeros((npad,), jnp.int32)])
    dst_p = jnp.concatenate(
        [ei[1], N + (jnp.arange(npad, dtype=jnp.int32) % (N_PAD - N))])
    dst_p = dst_p.reshape(E_PAD // C, C)

    parts1, cnt1 = _sc_aggregate(True)(x, src_p, dst_p)
    cnt = cnt1.reshape(NC, N_PAD)
    inv = (1.0 / jnp.maximum(cnt[0, :N] + cnt[1, :N], 1.0)).reshape(N, 1)
    h = _dense(parts1, inv, x, W1l, b1l, W1r, True)
    parts2, _ = _sc_aggregate(False)(h, src_p, dst_p)
    return _dense(parts2, inv, h, W2l, b2l, W2r, False)
